# baseline (speedup 1.0000x reference)
import functools
import math

import jax
import jax.numpy as jnp
from jax.experimental import pallas as pl
from jax.experimental.pallas import tpu as pltpu

_NZ = 100
_NL = 100000
_NU = 1024
_BK = 2048
_NBLK = (_NL + _BK - 1) // _BK
_KA = _NZ + 1
_LOG_NORM = 0.5 * _NZ * math.log(2.0 * math.pi)


def _dd_body(B_ref, A_ref, o_ref, tmin_ref):
    i = pl.program_id(0)
    Ab = A_ref[...]
    B = B_ref[...]
    t = jax.lax.dot_general(
        Ab, B.astype(jnp.bfloat16), (((1,), (0,)), ((), ())),
        preferred_element_type=jnp.float32,
    )

    @pl.when(i < _NBLK - 1)
    def _():
        bmin = jnp.min(t, axis=0, keepdims=True)
        tmin_ref[...] = jnp.where(i == 0, bmin,
                                  jnp.minimum(tmin_ref[...], bmin))

    @pl.when(i == _NBLK - 1)
    def _():
        gidx = i * _BK + jax.lax.broadcasted_iota(jnp.int32, (_BK, 1), 0)
        bmin = jnp.min(jnp.where(gidx < _NL, t, jnp.inf),
                       axis=0, keepdims=True)
        tmin = jnp.minimum(tmin_ref[...], bmin)
        U2 = 0.25 * jnp.sum(B[:_NZ, :] * B[:_NZ, :], axis=0,
                            keepdims=True)
        d2 = jnp.maximum(tmin + U2, 0.0)
        div = jnp.log(jnp.sqrt(d2) + 1e-18)
        dens = -0.5 * U2 - _LOG_NORM
        dd = jnp.exp(dens + div)
        dd = dd - jnp.min(dd)
        o_ref[...] = dd / (jnp.max(dd) + 1e-18)


@functools.partial(jax.jit, static_argnames=("interpret",))
def _dd_call(B, A, interpret=False):
    return pl.pallas_call(
        _dd_body,
        grid=(_NBLK,),
        in_specs=[
            pl.BlockSpec((_KA, _NU), lambda i: (0, 0)),
            pl.BlockSpec((_BK, _KA), lambda i: (i, 0)),
        ],
        out_specs=pl.BlockSpec((1, _NU), lambda i: (0, 0)),
        out_shape=jax.ShapeDtypeStruct((1, _NU), jnp.float32),
        scratch_shapes=[pltpu.VMEM((1, _NU), jnp.float32)],
        compiler_params=pltpu.CompilerParams(
            dimension_semantics=("arbitrary",),
        ),
        interpret=interpret,
    )(B, A)


def kernel(pred, U_z, L_z):
    del pred
    L2 = jnp.sum(L_z * L_z, axis=1, keepdims=True)
    A = jnp.concatenate([L_z, L2], axis=1).astype(jnp.bfloat16)
    B = jnp.concatenate(
        [-2.0 * U_z.T, jnp.ones((1, _NU), jnp.float32)], axis=0)
    out = _dd_call(B, A)
    return out.reshape(-1)
